# baseline (speedup 1.0000x reference)
import functools

import jax
import jax.numpy as jnp
from jax import lax
from jax.experimental import pallas as pl
from jax.experimental.pallas import tpu as pltpu
from jax.experimental.pallas import tpu_sc as plsc


@functools.lru_cache(maxsize=None)
def _sc_gather(vocab, embed, batch):
    info = plsc.get_sparse_core_info()
    nw = info.num_cores * info.num_subcores
    b_per_w = batch // nw
    mesh = plsc.VectorSubcoreMesh(core_axis_name="c", subcore_axis_name="s")

    @functools.partial(
        pl.kernel, mesh=mesh,
        out_type=jax.ShapeDtypeStruct((batch, embed), jnp.float32),
        scratch_types=[
            pltpu.VMEM((b_per_w,), jnp.int32),
            pltpu.VMEM((b_per_w, embed), jnp.float32),
            pltpu.SemaphoreType.DMA,
        ],
        compiler_params=pltpu.CompilerParams(use_tc_tiling_on_sc=False),
    )
    def gather_k(table_hbm, idx_hbm, out_hbm, idx_v, rows_v, sem):
        wid = lax.axis_index("s") * info.num_cores + lax.axis_index("c")
        base = wid * b_per_w
        pltpu.sync_copy(idx_hbm.at[pl.ds(base, b_per_w)], idx_v)
        pltpu.async_copy(table_hbm.at[idx_v], rows_v, sem).wait()
        pltpu.sync_copy(rows_v, out_hbm.at[pl.ds(base, b_per_w)])

    return gather_k


def _mm_body(h_ref, wt_ref, b_ref, o_ref):
    h = jnp.maximum(h_ref[...], 0.0).astype(jnp.bfloat16)
    acc = jnp.dot(h, wt_ref[...], preferred_element_type=jnp.float32)
    o_ref[...] = jnp.maximum(acc + b_ref[...], 0.0)


def _mm_body2(h_ref, wt_ref, b_ref, _prev_ref, o_ref):
    _mm_body(h_ref, wt_ref, b_ref, o_ref)


@functools.lru_cache(maxsize=None)
def _tc_matmul_half(batch, half, embed, out_dim, bt, row_blk_off):
    body = _mm_body if row_blk_off == 0 else _mm_body2
    in_specs = [
        pl.BlockSpec((bt, embed), lambda j: (j, 0)),
        pl.BlockSpec((embed, out_dim), lambda j: (0, 0)),
        pl.BlockSpec((1, out_dim), lambda j: (0, 0)),
    ]
    kwargs = {}
    if row_blk_off != 0:
        in_specs.append(pl.BlockSpec(memory_space=pl.ANY))
        kwargs["input_output_aliases"] = {3: 0}
    return pl.pallas_call(
        body,
        grid=(half // bt,),
        in_specs=in_specs,
        out_specs=pl.BlockSpec((bt, out_dim),
                               lambda j: (j + row_blk_off, 0)),
        out_shape=jax.ShapeDtypeStruct((batch, out_dim), jnp.float32),
        **kwargs,
    )


def kernel(x, emb, W, b):
    batch = x.shape[0]
    vocab, embed = emb.shape
    out_dim = W.shape[0]
    half = batch // 2
    bt = 64
    gather = _sc_gather(vocab, embed, half)
    h0 = gather(emb, x[:half])
    h1 = gather(emb, x[half:])
    wt = W.T.astype(jnp.bfloat16)
    b2 = b.reshape(1, out_dim)
    out = _tc_matmul_half(batch, half, embed, out_dim, bt, 0)(h0, wt, b2)
    return _tc_matmul_half(batch, half, embed, out_dim, bt, half // bt)(
        h1, wt, b2, out)


# speedup vs baseline: 1.0317x; 1.0317x over previous
import functools

import jax
import jax.numpy as jnp
from jax import lax
from jax.experimental import pallas as pl
from jax.experimental.pallas import tpu as pltpu
from jax.experimental.pallas import tpu_sc as plsc


@functools.lru_cache(maxsize=None)
def _sc_gather(vocab, embed, batch):
    info = plsc.get_sparse_core_info()
    nw = info.num_cores * info.num_subcores
    b_per_w = batch // nw
    mesh = plsc.VectorSubcoreMesh(core_axis_name="c", subcore_axis_name="s")

    @functools.partial(
        pl.kernel, mesh=mesh,
        out_type=jax.ShapeDtypeStruct((batch, embed), jnp.float32),
        scratch_types=[
            pltpu.VMEM((b_per_w,), jnp.int32),
            pltpu.VMEM((b_per_w, embed), jnp.float32),
            pltpu.SemaphoreType.DMA,
        ],
        compiler_params=pltpu.CompilerParams(use_tc_tiling_on_sc=False),
    )
    def gather_k(table_hbm, idx_hbm, out_hbm, idx_v, rows_v, sem):
        wid = lax.axis_index("s") * info.num_cores + lax.axis_index("c")
        base = wid * b_per_w
        pltpu.sync_copy(idx_hbm.at[pl.ds(base, b_per_w)], idx_v)
        pltpu.async_copy(table_hbm.at[idx_v], rows_v, sem).wait()
        pltpu.sync_copy(rows_v, out_hbm.at[pl.ds(base, b_per_w)])

    return gather_k


def _mm_body(h_ref, wt_ref, b_ref, o_ref):
    h = jnp.maximum(h_ref[...], 0.0).astype(jnp.bfloat16)
    acc = jnp.dot(h, wt_ref[...], preferred_element_type=jnp.float32)
    o_ref[...] = jnp.maximum(acc + b_ref[...], 0.0)


@functools.lru_cache(maxsize=None)
def _tc_matmul(batch, embed, out_dim, bt):
    return pl.pallas_call(
        _mm_body,
        grid=(batch // bt,),
        in_specs=[
            pl.BlockSpec((bt, embed), lambda j: (j, 0)),
            pl.BlockSpec((embed, out_dim), lambda j: (0, 0)),
            pl.BlockSpec((1, out_dim), lambda j: (0, 0)),
        ],
        out_specs=pl.BlockSpec((bt, out_dim), lambda j: (j, 0)),
        out_shape=jax.ShapeDtypeStruct((batch, out_dim), jnp.float32),
    )


def kernel(x, emb, W, b):
    batch = x.shape[0]
    vocab, embed = emb.shape
    out_dim = W.shape[0]
    h = _sc_gather(vocab, embed, batch)(emb, x)
    wt = W.T.astype(jnp.bfloat16)
    return _tc_matmul(batch, embed, out_dim, 32)(h, wt, b.reshape(1, out_dim))
